# Initial kernel scaffold; baseline (speedup 1.0000x reference)
import functools

import jax
import jax.numpy as jnp
from jax import lax
from jax.experimental import pallas as pl
from jax.experimental.pallas import tpu as pltpu
from jax.experimental.pallas import tpu_sc as plsc

BATCH = 4096
SEQ = 200
D = 64
NW = 32
ROWS = BATCH * SEQ
ROWS_PER_W = ROWS // NW
CHUNK = 128
CHUNKS_PER_W = ROWS_PER_W // CHUNK

_mesh = plsc.VectorSubcoreMesh(core_axis_name="c", subcore_axis_name="s")


@functools.partial(
    pl.kernel,
    out_type=jax.ShapeDtypeStruct((ROWS, D), jnp.float32),
    mesh=_mesh,
    scratch_types=[
        pltpu.VMEM((CHUNK,), jnp.int32),
        pltpu.VMEM((CHUNK, D), jnp.float32),
        pltpu.VMEM((2 * SEQ, D), jnp.float32),
        pltpu.SemaphoreType.DMA,
    ],
)
def _seq_embed(seq_hbm, tok_hbm, pos_hbm, out_hbm, idx_v, rows_v, pos_v, sem):
    wid = lax.axis_index("s") * 2 + lax.axis_index("c")
    base = wid * ROWS_PER_W
    pltpu.sync_copy(pos_hbm, pos_v.at[pl.ds(0, SEQ)])
    pltpu.sync_copy(pos_hbm, pos_v.at[pl.ds(SEQ, SEQ)])

    def chunk_body(g, _):
        row0 = base + g * CHUNK
        phase = (g * CHUNK) % SEQ
        pltpu.sync_copy(seq_hbm.at[pl.ds(row0, CHUNK)], idx_v)
        pltpu.async_copy(tok_hbm.at[idx_v], rows_v, sem).wait()

        def row_body(i, _):
            p = phase + i
            for j in range(D // 16):
                sl = pl.ds(j * 16, 16)
                plsc.addupdate(rows_v.at[i, sl], pos_v[p, sl])
            return 0

        lax.fori_loop(0, CHUNK, row_body, 0)
        pltpu.sync_copy(rows_v, out_hbm.at[pl.ds(row0, CHUNK)])
        return 0

    lax.fori_loop(0, CHUNKS_PER_W, chunk_body, 0)


def kernel(seq, token_table, pos_table):
    out = _seq_embed(seq.reshape(ROWS), token_table, pos_table)
    return out.reshape(BATCH, SEQ, D)


# speedup vs baseline: 2.2023x; 2.2023x over previous
import functools

import jax
import jax.numpy as jnp
from jax import lax
from jax.experimental import pallas as pl
from jax.experimental.pallas import tpu as pltpu
from jax.experimental.pallas import tpu_sc as plsc

BATCH = 4096
SEQ = 200
D = 64
NW = 32
ROWS = BATCH * SEQ
ROWS_PER_W = ROWS // NW
CHUNK = 128
CHUNKS_PER_W = ROWS_PER_W // CHUNK

_mesh = plsc.VectorSubcoreMesh(core_axis_name="c", subcore_axis_name="s")


@functools.partial(
    pl.kernel,
    out_type=jax.ShapeDtypeStruct((ROWS, D), jnp.float32),
    mesh=_mesh,
    scratch_types=[
        pltpu.VMEM((CHUNK,), jnp.int32),
        pltpu.VMEM((CHUNK, D), jnp.float32),
        pltpu.VMEM((2 * SEQ, D), jnp.float32),
        pltpu.SemaphoreType.DMA,
    ],
    compiler_params=pltpu.CompilerParams(use_tc_tiling_on_sc=False),
)
def _seq_embed(seq_hbm, tok_hbm, pos_hbm, out_hbm, idx_v, rows_v, pos_v, sem):
    wid = lax.axis_index("s") * 2 + lax.axis_index("c")
    base = wid * ROWS_PER_W
    pltpu.sync_copy(pos_hbm, pos_v.at[pl.ds(0, SEQ)])
    pltpu.sync_copy(pos_hbm, pos_v.at[pl.ds(SEQ, SEQ)])

    def chunk_body(g, _):
        row0 = base + g * CHUNK
        phase = (g * CHUNK) % SEQ
        pltpu.sync_copy(seq_hbm.at[pl.ds(row0, CHUNK)], idx_v)
        pltpu.async_copy(tok_hbm.at[idx_v], rows_v, sem).wait()

        def row_body(i, _):
            p = phase + i
            for j in range(D // 16):
                sl = pl.ds(j * 16, 16)
                plsc.addupdate(rows_v.at[i, sl], pos_v[p, sl])
            return 0

        lax.fori_loop(0, CHUNK, row_body, 0)
        pltpu.sync_copy(rows_v, out_hbm.at[pl.ds(row0, CHUNK)])
        return 0

    lax.fori_loop(0, CHUNKS_PER_W, chunk_body, 0)


def kernel(seq, token_table, pos_table):
    out = _seq_embed(seq.reshape(ROWS), token_table, pos_table)
    return out.reshape(BATCH, SEQ, D)


# speedup vs baseline: 2.7572x; 1.2519x over previous
import functools

import jax
import jax.numpy as jnp
from jax import lax
from jax.experimental import pallas as pl
from jax.experimental.pallas import tpu as pltpu
from jax.experimental.pallas import tpu_sc as plsc

BATCH = 4096
SEQ = 200
D = 64
NW = 32
ROWS = BATCH * SEQ
ROWS_PER_W = ROWS // NW
CHUNK = 128
CPW = ROWS_PER_W // CHUNK
NIN = 4
NOUT = 2
AHEAD = 3

_mesh = plsc.VectorSubcoreMesh(core_axis_name="c", subcore_axis_name="s")


@functools.partial(
    pl.kernel,
    out_type=jax.ShapeDtypeStruct((ROWS, D), jnp.float32),
    mesh=_mesh,
    scratch_types=[
        pltpu.VMEM((CPW, CHUNK), jnp.int32),
        pltpu.VMEM((2 * SEQ, D), jnp.float32),
        [pltpu.VMEM((CHUNK, D), jnp.float32) for _ in range(NIN)],
        [pltpu.VMEM((CHUNK, D), jnp.float32) for _ in range(NOUT)],
        [pltpu.SemaphoreType.DMA for _ in range(NIN)],
        [pltpu.SemaphoreType.DMA for _ in range(NOUT)],
    ],
    compiler_params=pltpu.CompilerParams(use_tc_tiling_on_sc=False),
)
def _seq_embed(seq_hbm, tok_hbm, pos_hbm, out_hbm,
               idx_v, pos_v, in_bufs, out_bufs, sem_in, sem_out):
    wid = lax.axis_index("s") * 2 + lax.axis_index("c")
    base = wid * ROWS_PER_W

    pltpu.sync_copy(seq_hbm.at[pl.ds(wid * CPW, CPW)], idx_v)
    pltpu.sync_copy(pos_hbm, pos_v.at[pl.ds(0, SEQ)])
    pltpu.sync_copy(pos_hbm, pos_v.at[pl.ds(SEQ, SEQ)])

    def issue_gather(g, bi):
        pltpu.make_async_copy(
            tok_hbm.at[idx_v.at[g]], in_bufs[bi], sem_in[bi]).start()

    def wait_gather(g, bi):
        pltpu.make_async_copy(
            tok_hbm.at[idx_v.at[g]], in_bufs[bi], sem_in[bi]).wait()

    def issue_out(g, bo):
        pltpu.make_async_copy(
            out_bufs[bo], out_hbm.at[pl.ds(base + g * CHUNK, CHUNK)],
            sem_out[bo]).start()

    def wait_out(g, bo):
        pltpu.make_async_copy(
            out_bufs[bo], out_hbm.at[pl.ds(base + g * CHUNK, CHUNK)],
            sem_out[bo]).wait()

    def add_chunk(g, bi, bo):
        phase = lax.rem(g * CHUNK, SEQ)

        @pl.loop(0, CHUNK, step=4)
        def _row(i):
            for r in range(4):
                p = phase + i + r
                for j in range(D // 16):
                    sl = pl.ds(j * 16, 16)
                    out_bufs[bo][i + r, sl] = in_bufs[bi][i + r, sl] + pos_v[p, sl]

    for g in range(AHEAD):
        issue_gather(g, g % NIN)

    def step(g, bi, bo, first):
        wait_gather(g, bi)
        if not first:
            wait_out(g - NOUT, bo)
        add_chunk(g, bi, bo)
        issue_out(g, bo)

    for g in range(NIN):
        step(g, g % NIN, g % NOUT, first=g < NOUT)
        issue_gather(g + AHEAD, (g + AHEAD) % NIN)

    @pl.loop(NIN, CPW, step=NIN)
    def _outer(go):
        for b in range(NIN):
            g = go + b
            step(g, b % NIN, b % NOUT, first=False)

            @pl.when(g + AHEAD < CPW)
            def _():
                issue_gather(g + AHEAD, (b + AHEAD) % NIN)

    for g in range(CPW - NOUT, CPW):
        wait_out(g, g % NOUT)


def kernel(seq, token_table, pos_table):
    out = _seq_embed(seq.reshape(ROWS // CHUNK, CHUNK), token_table, pos_table)
    return out.reshape(BATCH, SEQ, D)


# speedup vs baseline: 3.6003x; 1.3058x over previous
import functools

import jax
import jax.numpy as jnp
from jax import lax
from jax.experimental import pallas as pl
from jax.experimental.pallas import tpu as pltpu
from jax.experimental.pallas import tpu_sc as plsc

BATCH = 4096
SEQ = 200
D = 64
DP = 128
NW = 32
BPW = BATCH // NW
QPR = 5
QW = SEQ // QPR
NR = 4
NIN = 2 * QPR
NOUT = 2

_mesh = plsc.VectorSubcoreMesh(core_axis_name="c", subcore_axis_name="s")


@functools.partial(
    pl.kernel,
    out_type=jax.ShapeDtypeStruct((BATCH, SEQ, D), jnp.float32),
    mesh=_mesh,
    scratch_types=[
        [pltpu.VMEM((SEQ,), jnp.int32) for _ in range(NR)],
        pltpu.VMEM((SEQ // 2, DP), jnp.float32),
        [pltpu.VMEM((QW, DP), jnp.float32) for _ in range(NIN)],
        [pltpu.VMEM((SEQ, D), jnp.float32) for _ in range(NOUT)],
        [pltpu.SemaphoreType.DMA for _ in range(NR)],
        [pltpu.SemaphoreType.DMA for _ in range(NIN)],
        [pltpu.SemaphoreType.DMA for _ in range(NOUT)],
    ],
)
def _seq_embed(seq_hbm, tok_hbm, pos_hbm, out_hbm,
               idx_rings, pos_v, in_bufs, out_bufs, sem_idx, sem_in, sem_out):
    wid = lax.axis_index("s") * 2 + lax.axis_index("c")
    flat_base = wid * BPW * SEQ
    row_base = wid * BPW

    pltpu.sync_copy(pos_hbm, pos_v)

    def idx_desc(wr, jr):
        return pltpu.make_async_copy(
            seq_hbm.at[pl.ds(flat_base + wr * SEQ, SEQ)],
            idx_rings[jr], sem_idx[jr])

    def gather_desc(wr, jr, q, bi):
        idx = idx_rings[jr].at[pl.ds(q * QW, QW)]
        return pltpu.make_async_copy(tok_hbm.at[idx], in_bufs[bi], sem_in[bi])

    def out_desc(wr, bo):
        return pltpu.make_async_copy(
            out_bufs[bo], out_hbm.at[row_base + wr], sem_out[bo])

    def add_q(q, bi, bo):
        s = q * QW

        @pl.loop(0, QW // 4)
        def _i4(i4):
            for r in range(4):
                row = i4 * 4 + r
                prow = s // 2 + i4 * 2 + r // 2
                pcol = (r % 2) * D
                for j in range(D // 16):
                    sl = pl.ds(j * 16, 16)
                    out_bufs[bo][s + row, sl] = (
                        in_bufs[bi][row, sl] + pos_v[prow, pl.ds(pcol + j * 16, 16)])

    def row_body(wr, rr, *, outwait=True, idxwait=True, stage=True, nxt=True):
        bo = rr % NOUT
        if outwait:
            out_desc(wr - NOUT, bo).wait()
        if idxwait:
            idx_desc(wr + 1, (rr + 1) % NR).wait()
        if stage:
            idx_desc(wr + 2, (rr + 2) % NR).start()
        now = (rr % 2) * QPR
        nx = ((rr + 1) % 2) * QPR
        for q in range(QPR):
            gather_desc(wr, rr % NR, q, now + q).wait()
            if nxt:
                gather_desc(wr + 1, (rr + 1) % NR, q, nx + q).start()
            add_q(q, now + q, bo)
        out_desc(wr, bo).start()

    idx_desc(0, 0).start()
    idx_desc(1, 1).start()
    idx_desc(0, 0).wait()
    for q in range(QPR):
        gather_desc(0, 0, q, q).start()
    row_body(0, 0, outwait=False)
    row_body(1, 1, outwait=False)

    @pl.loop(0, (BPW - 4) // 4)
    def _outer(o):
        wr0 = 2 + o * 4
        for r in range(4):
            row_body(wr0 + r, 2 + r)

    row_body(BPW - 2, (BPW - 2) % 4, stage=False)
    row_body(BPW - 1, (BPW - 1) % 4, idxwait=False, stage=False, nxt=False)
    for wr in range(BPW - NOUT, BPW):
        out_desc(wr, wr % NOUT).wait()


def kernel(seq, token_table, pos_table):
    tok_p = jnp.pad(token_table, ((0, 0), (0, DP - D)))
    pos_p = jnp.concatenate([pos_table[0::2], pos_table[1::2]], axis=1)
    return _seq_embed(seq.reshape(BATCH * SEQ), tok_p, pos_p)


# speedup vs baseline: 4.9010x; 1.3613x over previous
import functools

import jax
import jax.numpy as jnp
from jax import lax
from jax.experimental import pallas as pl
from jax.experimental.pallas import tpu as pltpu
from jax.experimental.pallas import tpu_sc as plsc

BATCH = 4096
SEQ = 200
D = 64
DP = 128
NW = 32
ROWS = BATCH * SEQ
RPW = ROWS // NW
CHUNK = 128
CPW = RPW // CHUNK
BLK = 25
NIN = 5
AHEAD = 3

_mesh = plsc.VectorSubcoreMesh(core_axis_name="c", subcore_axis_name="s")


@functools.partial(
    pl.kernel,
    out_type=jax.ShapeDtypeStruct((ROWS, DP), jnp.float32),
    mesh=_mesh,
    scratch_types=[
        pltpu.VMEM((CPW, CHUNK), jnp.int32),
        pltpu.VMEM((SEQ // 2, DP), jnp.float32),
        [pltpu.VMEM((CHUNK, DP), jnp.float32) for _ in range(NIN)],
        pltpu.SemaphoreType.DMA,
        [pltpu.SemaphoreType.DMA for _ in range(NIN)],
        [pltpu.SemaphoreType.DMA for _ in range(NIN)],
    ],
)
def _seq_embed(seq_hbm, tok_hbm, pos_hbm, out_hbm,
               idx_v, pos_v, in_bufs, sem_pos, sem_in, sem_out):
    wid = lax.axis_index("s") * 2 + lax.axis_index("c")
    flat_base = wid * RPW

    pltpu.sync_copy(seq_hbm.at[pl.ds(wid * CPW, CPW)], idx_v)
    pltpu.sync_copy(pos_hbm, pos_v)

    def gather_desc(g, bi):
        return pltpu.make_async_copy(
            tok_hbm.at[idx_v.at[g]], in_bufs[bi], sem_in[bi])

    def out_desc(g, bi):
        return pltpu.make_async_copy(
            in_bufs[bi], out_hbm.at[pl.ds(flat_base + g * CHUNK, CHUNK)],
            sem_out[bi])

    def add_seg(bi, row0, nrows, phase):
        @pl.loop(0, nrows // 4)
        def _i4(i4):
            for r in range(4):
                prow = phase // 2 + i4 * 2 + r // 2
                pcol = (r % 2) * D
                for j in range(D // 16):
                    pvec = pos_v[prow, pl.ds(pcol + j * 16, 16)]
                    plsc.addupdate(
                        in_bufs[bi].at[row0 + i4 * 4 + r, pl.ds(j * 16, 16)],
                        pvec)

    for g in range(AHEAD):
        gather_desc(g, g % NIN).start()

    @pl.loop(0, CPW // BLK)
    def _blk(blk):
        g0 = blk * BLK
        for k in range(BLK):
            g = g0 + k
            bi = k % NIN
            b3 = (k + AHEAD) % NIN
            phase = (k * CHUNK) % SEQ
            l1 = min(CHUNK, SEQ - phase)
            gather_desc(g, bi).wait()

            @pl.when(g + AHEAD < CPW)
            def _(g=g, b3=b3):
                @pl.when(g >= NIN - AHEAD)
                def _():
                    out_desc(g - (NIN - AHEAD), b3).wait()

                gather_desc(g + AHEAD, b3).start()

            add_seg(bi, 0, l1, phase)
            if l1 < CHUNK:
                add_seg(bi, l1, CHUNK - l1, 0)
            out_desc(g, bi).start()

    for g in range(CPW - NIN, CPW):
        out_desc(g, g % NIN).wait()


def kernel(seq, token_table, pos_table):
    tok_p = jnp.pad(token_table, ((0, 0), (0, DP - D)))
    pos_p = jnp.concatenate([pos_table[0::2], pos_table[1::2]], axis=1)
    out = _seq_embed(seq.reshape(ROWS // CHUNK, CHUNK), tok_p, pos_p)
    return out[:, :D].reshape(BATCH, SEQ, D)
